# baseline (speedup 1.0000x reference)
import functools

import jax
import jax.numpy as jnp
from jax import lax
from jax.experimental import pallas as pl
from jax.experimental.pallas import tpu as pltpu
from jax.experimental.pallas import tpu_sc as plsc

N = 10000
E = 320000
D = 128
NC = 2
NS = 16
NW = NC * NS
EPT = E // NW
K = 80
NCHUNK = EPT // K
N_PAD = 10240
ROWS_PT = N_PAD // NS
ZROWS = 128
CNT_SLICE = 640
CNT_PAD = NS * CNT_SLICE
LAST = NCHUNK - 1

_mesh = plsc.VectorSubcoreMesh(core_axis_name="c", subcore_axis_name="s")


def _fill_const(ref, n, value):
    vec = jnp.full((16,), value, jnp.float32)

    def body(i, _):
        ref[pl.ds(i * 16, 16)] = vec
        return 0

    lax.fori_loop(0, n // 16, body, 0)


def _ring_setup(wid, hbm):
    dst3_hbm, src3_hbm, u_hbm, v_hbm, d_all, s_all, ur2, vr2, su, sv = hbm

    def issue_gather(ci, par):
        pltpu.async_copy(u_hbm.at[d_all.at[ci]], ur2.at[par], su)
        pltpu.async_copy(v_hbm.at[s_all.at[ci]], vr2.at[par], sv)

    def prologue():
        pltpu.sync_copy(dst3_hbm.at[wid], d_all)
        pltpu.sync_copy(src3_hbm.at[wid], s_all)
        issue_gather(0, 0)

    def step(i):
        p0 = lax.rem(i, 2)
        pltpu.make_async_copy(u_hbm.at[pl.ds(0, K)], ur2.at[0], su).wait()
        pltpu.make_async_copy(v_hbm.at[pl.ds(0, K)], vr2.at[0], sv).wait()
        issue_gather(jnp.minimum(i + 1, LAST), 1 - p0)
        return p0

    def drain():
        pltpu.make_async_copy(u_hbm.at[pl.ds(0, K)], ur2.at[0], su).wait()
        pltpu.make_async_copy(v_hbm.at[pl.ds(0, K)], vr2.at[0], sv).wait()

    return prologue, step, drain


@functools.partial(
    pl.kernel,
    out_type=(
        jax.ShapeDtypeStruct((NW, 2, D), jnp.float32),
        jax.ShapeDtypeStruct((NC, NS, CNT_SLICE), jnp.float32),
    ),
    mesh=_mesh,
    scratch_types=[
        pltpu.VMEM((NCHUNK, K), jnp.int32),
        pltpu.VMEM((NCHUNK, K), jnp.int32),
        pltpu.VMEM((2, K, D), jnp.float32),
        pltpu.VMEM((2, K, D), jnp.float32),
        pltpu.VMEM((K,), jnp.float32),
        pltpu.VMEM((CNT_SLICE,), jnp.float32),
        pltpu.VMEM((2, D), jnp.float32),
        pltpu.VMEM_SHARED((CNT_PAD,), jnp.float32),
        pltpu.SemaphoreType.DMA,
        pltpu.SemaphoreType.DMA,
    ],
)
def _sc_pass1(dst3_hbm, src3_hbm, u_hbm, v_hbm, stats_hbm, cnt_hbm,
              d_all, s_all, ur2, vr2, ones, zcnt, statsb, cnt_sh, su, sv):
    cid = lax.axis_index("c")
    sid = lax.axis_index("s")
    wid = sid * NC + cid
    base = wid * EPT

    _fill_const(ones, K, 1.0)
    _fill_const(zcnt, CNT_SLICE, 0.0)
    pltpu.sync_copy(zcnt, cnt_sh.at[pl.ds(sid * CNT_SLICE, CNT_SLICE)])
    plsc.subcore_barrier()

    prologue, ring_step, drain = _ring_setup(
        wid, (dst3_hbm, src3_hbm, u_hbm, v_hbm, d_all, s_all, ur2, vr2, su, sv))
    prologue()

    zero = jnp.zeros((16,), jnp.float32)
    init = tuple([zero] * 16)

    def chunk(i, carry):
        p0 = ring_step(i)

        def edge(e, car):
            new = list(car)
            for j in range(8):
                m = (ur2[p0, e, pl.ds(j * 16, 16)]
                     + vr2[p0, e, pl.ds(j * 16, 16)])
                new[j] = new[j] + m
                new[8 + j] = new[8 + j] + m * m
            return tuple(new)

        carry = lax.fori_loop(0, K, edge, carry)
        pltpu.sync_copy(ones, cnt_sh.at[d_all.at[i]], add=True)
        return carry

    acc = lax.fori_loop(0, NCHUNK, chunk, init)
    drain()

    for j in range(8):
        statsb[0, pl.ds(j * 16, 16)] = acc[j]
        statsb[1, pl.ds(j * 16, 16)] = acc[8 + j]
    pltpu.sync_copy(statsb, stats_hbm.at[wid])

    plsc.subcore_barrier()
    pltpu.sync_copy(cnt_sh.at[pl.ds(sid * CNT_SLICE, CNT_SLICE)],
                    cnt_hbm.at[cid, sid])


def _make_pass2(DH):
    JG = DH // 16

    @functools.partial(
        pl.kernel,
        out_type=jax.ShapeDtypeStruct((NC, N_PAD, DH), jnp.float32),
        mesh=_mesh,
        compiler_params=pltpu.CompilerParams(use_tc_tiling_on_sc=False),
        scratch_types=[
            pltpu.VMEM((2, K), jnp.int32),
            pltpu.VMEM((2, K), jnp.int32),
            pltpu.VMEM((2, K, DH), jnp.float32),
            pltpu.VMEM((2, K, DH), jnp.float32),
            pltpu.VMEM((2, DH), jnp.float32),
            pltpu.VMEM((ZROWS, DH), jnp.float32),
            pltpu.VMEM_SHARED((N_PAD, DH), jnp.float32),
            pltpu.SemaphoreType.DMA,
            pltpu.SemaphoreType.DMA,
        ],
    )
    def pass2h(dst_hbm, src_hbm, u_hbm, v_hbm, aff_hbm, out_hbm,
               d2, s2, ur2, vr2, afb, zb, out_sh, su, sv):
        cid = lax.axis_index("c")
        sid = lax.axis_index("s")
        base = (sid * NC + cid) * EPT

        pltpu.sync_copy(aff_hbm, afb)
        svec = [afb[0, pl.ds(j * 16, 16)] for j in range(JG)]
        tvec = [afb[1, pl.ds(j * 16, 16)] for j in range(JG)]

        zero = jnp.zeros((16,), jnp.float32)

        def zfill(i, _):
            zb[i // JG, pl.ds((i % JG) * 16, 16)] = zero
            return 0

        lax.fori_loop(0, ZROWS * JG, zfill, 0)
        for r in range(ROWS_PT // ZROWS):
            pltpu.sync_copy(zb, out_sh.at[pl.ds(sid * ROWS_PT + r * ZROWS, ZROWS)])
        plsc.subcore_barrier()

        def load_idx(ci, row):
            off = base + ci * K
            pltpu.sync_copy(dst_hbm.at[pl.ds(off, K)], d2.at[row])
            pltpu.sync_copy(src_hbm.at[pl.ds(off, K)], s2.at[row])

        def issue_gather(row):
            pltpu.async_copy(u_hbm.at[d2.at[row]], ur2.at[row], su)
            pltpu.async_copy(v_hbm.at[s2.at[row]], vr2.at[row], sv)

        def wait_gather():
            pltpu.make_async_copy(u_hbm.at[pl.ds(0, K)], ur2.at[0], su).wait()
            pltpu.make_async_copy(v_hbm.at[pl.ds(0, K)], vr2.at[0], sv).wait()

        load_idx(0, 0)
        issue_gather(0)

        def chunk(i, _):
            p0 = lax.rem(i, 2)
            p1 = 1 - p0
            wait_gather()
            load_idx(jnp.minimum(i + 1, LAST), p1)
            issue_gather(p1)

            def edge(e, _):
                for j in range(JG):
                    m = (ur2[p0, e, pl.ds(j * 16, 16)]
                         + vr2[p0, e, pl.ds(j * 16, 16)])
                    y = m * svec[j] + tvec[j]
                    ur2[p0, e, pl.ds(j * 16, 16)] = jnp.maximum(y, y * 0.3)
                return 0

            lax.fori_loop(0, K, edge, 0)
            pltpu.sync_copy(ur2.at[p0], out_sh.at[d2.at[p0]], add=True)
            return 0

        lax.fori_loop(0, NCHUNK, chunk, 0)
        wait_gather()
        plsc.subcore_barrier()

        for r in range(ROWS_PT // ZROWS):
            rows = pl.ds(sid * ROWS_PT + r * ZROWS, ZROWS)
            pltpu.sync_copy(out_sh.at[rows], out_hbm.at[cid, rows])

    return pass2h


_sc_pass2h = _make_pass2(D // 2)


def _tc_uv(x, wcomb):
    H = D // 2

    def body(x_ref, w_ref, u_ref, v_ref, ul_ref, uh_ref, vl_ref, vh_ref):
        xv = x_ref[...]
        u = jnp.dot(xv, w_ref[:, :D], preferred_element_type=jnp.float32)
        v = jnp.dot(xv, w_ref[:, D:], preferred_element_type=jnp.float32)
        u_ref[...] = u
        v_ref[...] = v
        ul_ref[...] = u[:, :H]
        uh_ref[...] = u[:, H:]
        vl_ref[...] = v[:, :H]
        vh_ref[...] = v[:, H:]

    full = jax.ShapeDtypeStruct((N, D), jnp.float32)
    half = jax.ShapeDtypeStruct((N, H), jnp.float32)
    return pl.pallas_call(
        body,
        out_shape=(full, full, half, half, half, half),
    )(x, wcomb)


def _tc_epilogue(plo, phi, inv_cnt):
    H = D // 2

    def body(a_ref, b_ref, i_ref, o_ref):
        inv = i_ref[...]
        o_ref[:, :H] = (a_ref[0] + a_ref[1]) * inv
        o_ref[:, H:] = (b_ref[0] + b_ref[1]) * inv

    return pl.pallas_call(
        body,
        out_shape=jax.ShapeDtypeStruct((N, D), jnp.float32),
    )(plo, phi, inv_cnt)


def kernel(feature, edge_index, W, b, gamma, beta):
    del b
    dst = edge_index[1].astype(jnp.int32)
    src = edge_index[0].astype(jnp.int32)
    dst3 = dst.reshape(NW, NCHUNK, K)
    src3 = src.reshape(NW, NCHUNK, K)

    w1 = W[:, :D]
    w2 = W[:, D:]
    wcomb = jnp.concatenate([(w1 - w2).T, w2.T], axis=1)

    u, v, ul, uh, vl, vh = _tc_uv(feature, wcomb)

    stats, cntp = _sc_pass1(dst3, src3, u, v)
    ssum = stats[:, 0, :].sum(axis=0)
    sqsum = stats[:, 1, :].sum(axis=0)
    mean = ssum / E
    var = sqsum / E - mean * mean
    scale = gamma * lax.rsqrt(var + 1e-5)
    shift = beta - mean * scale
    aff = jnp.stack([scale, shift])

    cnt = cntp.reshape(NC, CNT_PAD)[:, :N].sum(axis=0)
    inv_cnt = (1.0 / jnp.maximum(cnt, 1.0)).reshape(N, 1)

    h = D // 2
    plo = _sc_pass2h(dst, src, ul, vl, aff[:, :h])[:, :N, :]
    phi = _sc_pass2h(dst, src, uh, vh, aff[:, h:])[:, :N, :]
    return _tc_epilogue(plo, phi, inv_cnt)


# speedup vs baseline: 2.3219x; 2.3219x over previous
import functools

import jax
import jax.numpy as jnp
from jax import lax
from jax.experimental import pallas as pl
from jax.experimental.pallas import tpu as pltpu
from jax.experimental.pallas import tpu_sc as plsc

N = 10000
E = 320000
D = 128
NC = 2
NS = 16
NW = NC * NS
EPT = E // NW
K = 80
NCHUNK = EPT // K
N_PAD = 10240
ROWS_PT = N_PAD // NS
ZROWS = 128
CNT_SLICE = 640
CNT_PAD = NS * CNT_SLICE
LAST = NCHUNK - 1

_mesh = plsc.VectorSubcoreMesh(core_axis_name="c", subcore_axis_name="s")


def _fill_const(ref, n, value):
    vec = jnp.full((16,), value, jnp.float32)

    def body(i, _):
        ref[pl.ds(i * 16, 16)] = vec
        return 0

    lax.fori_loop(0, n // 16, body, 0)


def _ring_setup(wid, hbm):
    dst3_hbm, src3_hbm, u_hbm, v_hbm, d_all, s_all, ur2, vr2, su, sv = hbm

    def issue_gather(ci, par):
        pltpu.async_copy(u_hbm.at[d_all.at[ci]], ur2.at[par], su)
        pltpu.async_copy(v_hbm.at[s_all.at[ci]], vr2.at[par], sv)

    def prologue():
        pltpu.sync_copy(dst3_hbm.at[wid], d_all)
        pltpu.sync_copy(src3_hbm.at[wid], s_all)
        issue_gather(0, 0)

    def step(i):
        p0 = lax.rem(i, 2)
        pltpu.make_async_copy(u_hbm.at[pl.ds(0, K)], ur2.at[0], su).wait()
        pltpu.make_async_copy(v_hbm.at[pl.ds(0, K)], vr2.at[0], sv).wait()
        issue_gather(jnp.minimum(i + 1, LAST), 1 - p0)
        return p0

    def drain():
        pltpu.make_async_copy(u_hbm.at[pl.ds(0, K)], ur2.at[0], su).wait()
        pltpu.make_async_copy(v_hbm.at[pl.ds(0, K)], vr2.at[0], sv).wait()

    return prologue, step, drain


@functools.partial(
    pl.kernel,
    out_type=(
        jax.ShapeDtypeStruct((NW, 2, D), jnp.float32),
        jax.ShapeDtypeStruct((NC, NS, CNT_SLICE), jnp.float32),
    ),
    mesh=_mesh,
    scratch_types=[
        pltpu.VMEM((NCHUNK, K), jnp.int32),
        pltpu.VMEM((NCHUNK, K), jnp.int32),
        pltpu.VMEM((2, K, D), jnp.float32),
        pltpu.VMEM((2, K, D), jnp.float32),
        pltpu.VMEM((K,), jnp.float32),
        pltpu.VMEM((CNT_SLICE,), jnp.float32),
        pltpu.VMEM((2, D), jnp.float32),
        pltpu.VMEM_SHARED((CNT_PAD,), jnp.float32),
        pltpu.SemaphoreType.DMA,
        pltpu.SemaphoreType.DMA,
    ],
)
def _sc_pass1(dst3_hbm, src3_hbm, u_hbm, v_hbm, stats_hbm, cnt_hbm,
              d_all, s_all, ur2, vr2, ones, zcnt, statsb, cnt_sh, su, sv):
    cid = lax.axis_index("c")
    sid = lax.axis_index("s")
    wid = sid * NC + cid
    base = wid * EPT

    _fill_const(ones, K, 1.0)
    _fill_const(zcnt, CNT_SLICE, 0.0)
    pltpu.sync_copy(zcnt, cnt_sh.at[pl.ds(sid * CNT_SLICE, CNT_SLICE)])
    plsc.subcore_barrier()

    prologue, ring_step, drain = _ring_setup(
        wid, (dst3_hbm, src3_hbm, u_hbm, v_hbm, d_all, s_all, ur2, vr2, su, sv))
    prologue()

    zero = jnp.zeros((16,), jnp.float32)
    init = tuple([zero] * 16)

    def chunk(i, carry):
        p0 = ring_step(i)

        def edge(e, car):
            new = list(car)
            for j in range(8):
                m = (ur2[p0, e, pl.ds(j * 16, 16)]
                     + vr2[p0, e, pl.ds(j * 16, 16)])
                new[j] = new[j] + m
                new[8 + j] = new[8 + j] + m * m
            return tuple(new)

        carry = lax.fori_loop(0, K, edge, carry)
        pltpu.sync_copy(ones, cnt_sh.at[d_all.at[i]], add=True)
        return carry

    acc = lax.fori_loop(0, NCHUNK, chunk, init)
    drain()

    for j in range(8):
        statsb[0, pl.ds(j * 16, 16)] = acc[j]
        statsb[1, pl.ds(j * 16, 16)] = acc[8 + j]
    pltpu.sync_copy(statsb, stats_hbm.at[wid])

    plsc.subcore_barrier()
    pltpu.sync_copy(cnt_sh.at[pl.ds(sid * CNT_SLICE, CNT_SLICE)],
                    cnt_hbm.at[cid, sid])


@functools.partial(
    pl.kernel,
    out_type=jax.ShapeDtypeStruct((NC, N_PAD, D), jnp.float32),
    mesh=_mesh,
    scratch_types=[
        pltpu.VMEM((2, K), jnp.int32),
        pltpu.VMEM((2, K), jnp.int32),
        pltpu.VMEM((K, D), jnp.float32),
        pltpu.VMEM((K, D), jnp.float32),
        pltpu.VMEM((2, D), jnp.float32),
        pltpu.VMEM((ZROWS, D), jnp.float32),
        pltpu.VMEM_SHARED((N_PAD, D), jnp.float32),
        pltpu.SemaphoreType.DMA,
        pltpu.SemaphoreType.DMA,
    ],
)
def _sc_pass2(dst_hbm, src_hbm, u_hbm, v_hbm, aff_hbm, out_hbm,
              d2, s2, ur, vr, afb, zb, out_sh, su, sv):
    cid = lax.axis_index("c")
    sid = lax.axis_index("s")
    base = (sid * NC + cid) * EPT

    pltpu.sync_copy(aff_hbm, afb)
    svec = [afb[0, pl.ds(j * 16, 16)] for j in range(8)]
    tvec = [afb[1, pl.ds(j * 16, 16)] for j in range(8)]

    zero = jnp.zeros((16,), jnp.float32)

    def zfill(i, _):
        zb[i // 8, pl.ds((i % 8) * 16, 16)] = zero
        return 0

    lax.fori_loop(0, ZROWS * 8, zfill, 0)
    for r in range(ROWS_PT // ZROWS):
        pltpu.sync_copy(zb, out_sh.at[pl.ds(sid * ROWS_PT + r * ZROWS, ZROWS)])
    plsc.subcore_barrier()

    def load_idx(ci, row):
        off = base + ci * K
        pltpu.sync_copy(dst_hbm.at[pl.ds(off, K)], d2.at[row])
        pltpu.sync_copy(src_hbm.at[pl.ds(off, K)], s2.at[row])

    load_idx(0, 0)

    def chunk(i, _):
        p0 = lax.rem(i, 2)
        pltpu.async_copy(u_hbm.at[d2.at[p0]], ur, su)
        pltpu.async_copy(v_hbm.at[s2.at[p0]], vr, sv)
        load_idx(jnp.minimum(i + 1, LAST), 1 - p0)
        pltpu.make_async_copy(u_hbm.at[pl.ds(0, K)], ur, su).wait()
        pltpu.make_async_copy(v_hbm.at[pl.ds(0, K)], vr, sv).wait()

        def edge(e, _):
            for j in range(8):
                m = ur[e, pl.ds(j * 16, 16)] + vr[e, pl.ds(j * 16, 16)]
                y = m * svec[j] + tvec[j]
                ur[e, pl.ds(j * 16, 16)] = jnp.maximum(y, y * 0.3)
            return 0

        lax.fori_loop(0, K, edge, 0)
        pltpu.sync_copy(ur, out_sh.at[d2.at[p0]], add=True)
        return 0

    lax.fori_loop(0, NCHUNK, chunk, 0)
    plsc.subcore_barrier()

    for r in range(ROWS_PT // ZROWS):
        rows = pl.ds(sid * ROWS_PT + r * ZROWS, ZROWS)
        pltpu.sync_copy(out_sh.at[rows], out_hbm.at[cid, rows])


def _tc_uv(x, wcomb):
    def body(x_ref, w_ref, u_ref, v_ref):
        xv = x_ref[...]
        u_ref[...] = jnp.dot(xv, w_ref[:, :D], preferred_element_type=jnp.float32)
        v_ref[...] = jnp.dot(xv, w_ref[:, D:], preferred_element_type=jnp.float32)

    return pl.pallas_call(
        body,
        out_shape=(
            jax.ShapeDtypeStruct((N, D), jnp.float32),
            jax.ShapeDtypeStruct((N, D), jnp.float32),
        ),
    )(x, wcomb)


def _tc_epilogue(partials, inv_cnt):
    def body(p_ref, i_ref, o_ref):
        o_ref[...] = (p_ref[0] + p_ref[1]) * i_ref[...]

    return pl.pallas_call(
        body,
        out_shape=jax.ShapeDtypeStruct((N, D), jnp.float32),
    )(partials, inv_cnt)


def kernel(feature, edge_index, W, b, gamma, beta):
    del b
    dst = edge_index[1].astype(jnp.int32)
    src = edge_index[0].astype(jnp.int32)
    dst3 = dst.reshape(NW, NCHUNK, K)
    src3 = src.reshape(NW, NCHUNK, K)

    w1 = W[:, :D]
    w2 = W[:, D:]
    wcomb = jnp.concatenate([(w1 - w2).T, w2.T], axis=1)

    u, v = _tc_uv(feature, wcomb)

    stats, cntp = _sc_pass1(dst3, src3, u, v)
    ssum = stats[:, 0, :].sum(axis=0)
    sqsum = stats[:, 1, :].sum(axis=0)
    mean = ssum / E
    var = sqsum / E - mean * mean
    scale = gamma * lax.rsqrt(var + 1e-5)
    shift = beta - mean * scale
    aff = jnp.stack([scale, shift])

    cnt = cntp.reshape(NC, CNT_PAD)[:, :N].sum(axis=0)
    inv_cnt = (1.0 / jnp.maximum(cnt, 1.0)).reshape(N, 1)

    partials = _sc_pass2(dst, src, u, v, aff)[:, :N, :]
    return _tc_epilogue(partials, inv_cnt)


# speedup vs baseline: 2.3282x; 1.0027x over previous
import functools

import jax
import jax.numpy as jnp
from jax import lax
from jax.experimental import pallas as pl
from jax.experimental.pallas import tpu as pltpu
from jax.experimental.pallas import tpu_sc as plsc

N = 10000
E = 320000
D = 128
NC = 2
NS = 16
NW = NC * NS
EPT = E // NW
K = 80
NCHUNK = EPT // K
N_PAD = 10240
ROWS_PT = N_PAD // NS
ZROWS = 128
CNT_SLICE = 640
CNT_PAD = NS * CNT_SLICE
LAST = NCHUNK - 1

_mesh = plsc.VectorSubcoreMesh(core_axis_name="c", subcore_axis_name="s")


def _fill_const(ref, n, value):
    vec = jnp.full((16,), value, jnp.float32)

    def body(i, _):
        ref[pl.ds(i * 16, 16)] = vec
        return 0

    lax.fori_loop(0, n // 16, body, 0)


def _ring_setup(wid, hbm):
    dst3_hbm, src3_hbm, u_hbm, v_hbm, d_all, s_all, ur2, vr2, su, sv = hbm

    def issue_gather(ci, par):
        pltpu.async_copy(u_hbm.at[d_all.at[ci]], ur2.at[par], su)
        pltpu.async_copy(v_hbm.at[s_all.at[ci]], vr2.at[par], sv)

    def prologue():
        pltpu.sync_copy(dst3_hbm.at[wid], d_all)
        pltpu.sync_copy(src3_hbm.at[wid], s_all)
        issue_gather(0, 0)

    def step(i):
        p0 = lax.rem(i, 2)
        pltpu.make_async_copy(u_hbm.at[pl.ds(0, K)], ur2.at[0], su).wait()
        pltpu.make_async_copy(v_hbm.at[pl.ds(0, K)], vr2.at[0], sv).wait()
        issue_gather(jnp.minimum(i + 1, LAST), 1 - p0)
        return p0

    def drain():
        pltpu.make_async_copy(u_hbm.at[pl.ds(0, K)], ur2.at[0], su).wait()
        pltpu.make_async_copy(v_hbm.at[pl.ds(0, K)], vr2.at[0], sv).wait()

    return prologue, step, drain


@functools.partial(
    pl.kernel,
    out_type=(
        jax.ShapeDtypeStruct((NW, 2, D), jnp.float32),
        jax.ShapeDtypeStruct((NC, NS, CNT_SLICE), jnp.float32),
    ),
    mesh=_mesh,
    scratch_types=[
        pltpu.VMEM((NCHUNK, K), jnp.int32),
        pltpu.VMEM((NCHUNK, K), jnp.int32),
        pltpu.VMEM((2, K, D), jnp.float32),
        pltpu.VMEM((2, K, D), jnp.float32),
        pltpu.VMEM((K,), jnp.float32),
        pltpu.VMEM((CNT_SLICE,), jnp.float32),
        pltpu.VMEM((2, D), jnp.float32),
        pltpu.VMEM_SHARED((CNT_PAD,), jnp.float32),
        pltpu.SemaphoreType.DMA,
        pltpu.SemaphoreType.DMA,
    ],
)
def _sc_pass1(dst3_hbm, src3_hbm, u_hbm, v_hbm, stats_hbm, cnt_hbm,
              d_all, s_all, ur2, vr2, ones, zcnt, statsb, cnt_sh, su, sv):
    cid = lax.axis_index("c")
    sid = lax.axis_index("s")
    wid = sid * NC + cid
    base = wid * EPT

    _fill_const(ones, K, 1.0)
    _fill_const(zcnt, CNT_SLICE, 0.0)
    pltpu.sync_copy(zcnt, cnt_sh.at[pl.ds(sid * CNT_SLICE, CNT_SLICE)])
    plsc.subcore_barrier()

    prologue, ring_step, drain = _ring_setup(
        wid, (dst3_hbm, src3_hbm, u_hbm, v_hbm, d_all, s_all, ur2, vr2, su, sv))
    prologue()

    zero = jnp.zeros((16,), jnp.float32)
    init = tuple([zero] * 16)

    def chunk(i, carry):
        p0 = ring_step(i)

        def edge(e2, car):
            new = list(car)
            for de in range(2):
                e = e2 * 2 + de
                for j in range(8):
                    m = (ur2[p0, e, pl.ds(j * 16, 16)]
                         + vr2[p0, e, pl.ds(j * 16, 16)])
                    new[j] = new[j] + m
                    new[8 + j] = new[8 + j] + m * m
            return tuple(new)

        carry = lax.fori_loop(0, K // 2, edge, carry)
        pltpu.sync_copy(ones, cnt_sh.at[d_all.at[i]], add=True)
        return carry

    acc = lax.fori_loop(0, NCHUNK, chunk, init)
    drain()

    for j in range(8):
        statsb[0, pl.ds(j * 16, 16)] = acc[j]
        statsb[1, pl.ds(j * 16, 16)] = acc[8 + j]
    pltpu.sync_copy(statsb, stats_hbm.at[wid])

    plsc.subcore_barrier()
    pltpu.sync_copy(cnt_sh.at[pl.ds(sid * CNT_SLICE, CNT_SLICE)],
                    cnt_hbm.at[cid, sid])


@functools.partial(
    pl.kernel,
    out_type=jax.ShapeDtypeStruct((NC, N_PAD, D), jnp.float32),
    mesh=_mesh,
    scratch_types=[
        pltpu.VMEM((2, K), jnp.int32),
        pltpu.VMEM((2, K), jnp.int32),
        pltpu.VMEM((K, D), jnp.float32),
        pltpu.VMEM((K, D), jnp.float32),
        pltpu.VMEM((2, D), jnp.float32),
        pltpu.VMEM((ZROWS, D), jnp.float32),
        pltpu.VMEM_SHARED((N_PAD, D), jnp.float32),
        pltpu.SemaphoreType.DMA,
        pltpu.SemaphoreType.DMA,
    ],
)
def _sc_pass2(dst_hbm, src_hbm, u_hbm, v_hbm, aff_hbm, out_hbm,
              d2, s2, ur, vr, afb, zb, out_sh, su, sv):
    cid = lax.axis_index("c")
    sid = lax.axis_index("s")
    base = (sid * NC + cid) * EPT

    pltpu.sync_copy(aff_hbm, afb)
    svec = [afb[0, pl.ds(j * 16, 16)] for j in range(8)]
    tvec = [afb[1, pl.ds(j * 16, 16)] for j in range(8)]

    zero = jnp.zeros((16,), jnp.float32)

    def zfill(i, _):
        zb[i // 8, pl.ds((i % 8) * 16, 16)] = zero
        return 0

    lax.fori_loop(0, ZROWS * 8, zfill, 0)
    for r in range(ROWS_PT // ZROWS):
        pltpu.sync_copy(zb, out_sh.at[pl.ds(sid * ROWS_PT + r * ZROWS, ZROWS)])
    plsc.subcore_barrier()

    def load_idx(ci, row):
        off = base + ci * K
        pltpu.sync_copy(dst_hbm.at[pl.ds(off, K)], d2.at[row])
        pltpu.sync_copy(src_hbm.at[pl.ds(off, K)], s2.at[row])

    load_idx(0, 0)

    def chunk(i, _):
        p0 = lax.rem(i, 2)
        pltpu.async_copy(u_hbm.at[d2.at[p0]], ur, su)
        pltpu.async_copy(v_hbm.at[s2.at[p0]], vr, sv)
        load_idx(jnp.minimum(i + 1, LAST), 1 - p0)
        pltpu.make_async_copy(u_hbm.at[pl.ds(0, K)], ur, su).wait()
        pltpu.make_async_copy(v_hbm.at[pl.ds(0, K)], vr, sv).wait()

        def edge(e2, _):
            for de in range(2):
                e = e2 * 2 + de
                for j in range(8):
                    m = ur[e, pl.ds(j * 16, 16)] + vr[e, pl.ds(j * 16, 16)]
                    y = m * svec[j] + tvec[j]
                    ur[e, pl.ds(j * 16, 16)] = jnp.maximum(y, y * 0.3)
            return 0

        lax.fori_loop(0, K // 2, edge, 0)
        pltpu.sync_copy(ur, out_sh.at[d2.at[p0]], add=True)
        return 0

    lax.fori_loop(0, NCHUNK, chunk, 0)
    plsc.subcore_barrier()

    for r in range(ROWS_PT // ZROWS):
        rows = pl.ds(sid * ROWS_PT + r * ZROWS, ZROWS)
        pltpu.sync_copy(out_sh.at[rows], out_hbm.at[cid, rows])


def _tc_uv(x, wcomb):
    def body(x_ref, w_ref, u_ref, v_ref):
        xv = x_ref[...]
        u_ref[...] = jnp.dot(xv, w_ref[:, :D], preferred_element_type=jnp.float32)
        v_ref[...] = jnp.dot(xv, w_ref[:, D:], preferred_element_type=jnp.float32)

    return pl.pallas_call(
        body,
        out_shape=(
            jax.ShapeDtypeStruct((N, D), jnp.float32),
            jax.ShapeDtypeStruct((N, D), jnp.float32),
        ),
    )(x, wcomb)


def _tc_epilogue(partials, inv_cnt):
    def body(p_ref, i_ref, o_ref):
        o_ref[...] = (p_ref[0] + p_ref[1]) * i_ref[...]

    return pl.pallas_call(
        body,
        out_shape=jax.ShapeDtypeStruct((N, D), jnp.float32),
    )(partials, inv_cnt)


def kernel(feature, edge_index, W, b, gamma, beta):
    del b
    dst = edge_index[1].astype(jnp.int32)
    src = edge_index[0].astype(jnp.int32)
    dst3 = dst.reshape(NW, NCHUNK, K)
    src3 = src.reshape(NW, NCHUNK, K)

    w1 = W[:, :D]
    w2 = W[:, D:]
    wcomb = jnp.concatenate([(w1 - w2).T, w2.T], axis=1)

    u, v = _tc_uv(feature, wcomb)

    stats, cntp = _sc_pass1(dst3, src3, u, v)
    ssum = stats[:, 0, :].sum(axis=0)
    sqsum = stats[:, 1, :].sum(axis=0)
    mean = ssum / E
    var = sqsum / E - mean * mean
    scale = gamma * lax.rsqrt(var + 1e-5)
    shift = beta - mean * scale
    aff = jnp.stack([scale, shift])

    cnt = cntp.reshape(NC, CNT_PAD)[:, :N].sum(axis=0)
    inv_cnt = (1.0 / jnp.maximum(cnt, 1.0)).reshape(N, 1)

    partials = _sc_pass2(dst, src, u, v, aff)[:, :N, :]
    return _tc_epilogue(partials, inv_cnt)


# speedup vs baseline: 2.4535x; 1.0538x over previous
import functools

import jax
import jax.numpy as jnp
from jax import lax
from jax.experimental import pallas as pl
from jax.experimental.pallas import tpu as pltpu
from jax.experimental.pallas import tpu_sc as plsc

N = 10000
E = 320000
D = 128
NC = 2
NS = 16
NW = NC * NS
EPT = E // NW
K = 80
NCHUNK = EPT // K
N_PAD = 10240
ROWS_PT = N_PAD // NS
ZROWS = 128
CNT_SLICE = 640
CNT_PAD = NS * CNT_SLICE
LAST = NCHUNK - 1

_mesh = plsc.VectorSubcoreMesh(core_axis_name="c", subcore_axis_name="s")


def _fill_const(ref, n, value):
    vec = jnp.full((16,), value, jnp.float32)

    def body(i, _):
        ref[pl.ds(i * 16, 16)] = vec
        return 0

    lax.fori_loop(0, n // 16, body, 0)


def _ring_setup(wid, hbm):
    dst3_hbm, src3_hbm, u_hbm, v_hbm, d_all, s_all, ur2, vr2, su, sv = hbm

    def issue_gather(ci, par):
        pltpu.async_copy(u_hbm.at[d_all.at[ci]], ur2.at[par], su)
        pltpu.async_copy(v_hbm.at[s_all.at[ci]], vr2.at[par], sv)

    def prologue():
        pltpu.sync_copy(dst3_hbm.at[wid], d_all)
        pltpu.sync_copy(src3_hbm.at[wid], s_all)
        issue_gather(0, 0)

    def step(i):
        p0 = lax.rem(i, 2)
        pltpu.make_async_copy(u_hbm.at[pl.ds(0, K)], ur2.at[0], su).wait()
        pltpu.make_async_copy(v_hbm.at[pl.ds(0, K)], vr2.at[0], sv).wait()
        issue_gather(jnp.minimum(i + 1, LAST), 1 - p0)
        return p0

    def drain():
        pltpu.make_async_copy(u_hbm.at[pl.ds(0, K)], ur2.at[0], su).wait()
        pltpu.make_async_copy(v_hbm.at[pl.ds(0, K)], vr2.at[0], sv).wait()

    return prologue, step, drain


@functools.partial(
    pl.kernel,
    out_type=(
        jax.ShapeDtypeStruct((NW, 2, D), jnp.float32),
        jax.ShapeDtypeStruct((NC, NS, CNT_SLICE), jnp.float32),
    ),
    mesh=_mesh,
    scratch_types=[
        pltpu.VMEM((NCHUNK, K), jnp.int32),
        pltpu.VMEM((NCHUNK, K), jnp.int32),
        pltpu.VMEM((2, K, D), jnp.float32),
        pltpu.VMEM((2, K, D), jnp.float32),
        pltpu.VMEM((K,), jnp.float32),
        pltpu.VMEM((CNT_SLICE,), jnp.float32),
        pltpu.VMEM((2, D), jnp.float32),
        pltpu.VMEM_SHARED((CNT_PAD,), jnp.float32),
        pltpu.SemaphoreType.DMA,
        pltpu.SemaphoreType.DMA,
    ],
)
def _sc_pass1(dst3_hbm, src3_hbm, u_hbm, v_hbm, stats_hbm, cnt_hbm,
              d_all, s_all, ur2, vr2, ones, zcnt, statsb, cnt_sh, su, sv):
    cid = lax.axis_index("c")
    sid = lax.axis_index("s")
    wid = sid * NC + cid
    base = wid * EPT

    _fill_const(ones, K, 1.0)
    _fill_const(zcnt, CNT_SLICE, 0.0)
    pltpu.sync_copy(zcnt, cnt_sh.at[pl.ds(sid * CNT_SLICE, CNT_SLICE)])
    plsc.subcore_barrier()

    prologue, ring_step, drain = _ring_setup(
        wid, (dst3_hbm, src3_hbm, u_hbm, v_hbm, d_all, s_all, ur2, vr2, su, sv))
    prologue()

    zero = jnp.zeros((16,), jnp.float32)
    init = tuple([zero] * 16)

    def chunk(i, carry):
        p0 = ring_step(i)

        def edge(e2, car):
            new = list(car)
            for de in range(2):
                e = e2 * 2 + de
                for j in range(8):
                    m = (ur2[p0, e, pl.ds(j * 16, 16)]
                         + vr2[p0, e, pl.ds(j * 16, 16)])
                    new[j] = new[j] + m
                    new[8 + j] = new[8 + j] + m * m
            return tuple(new)

        carry = lax.fori_loop(0, K // 2, edge, carry)
        pltpu.sync_copy(ones, cnt_sh.at[d_all.at[i]], add=True)
        return carry

    acc = lax.fori_loop(0, NCHUNK, chunk, init)
    drain()

    for j in range(8):
        statsb[0, pl.ds(j * 16, 16)] = acc[j]
        statsb[1, pl.ds(j * 16, 16)] = acc[8 + j]
    pltpu.sync_copy(statsb, stats_hbm.at[wid])

    plsc.subcore_barrier()
    pltpu.sync_copy(cnt_sh.at[pl.ds(sid * CNT_SLICE, CNT_SLICE)],
                    cnt_hbm.at[cid, sid])


@functools.partial(
    pl.kernel,
    out_type=jax.ShapeDtypeStruct((NC, N_PAD, D), jnp.float32),
    mesh=_mesh,
    scratch_types=[
        pltpu.VMEM((2, K), jnp.int32),
        pltpu.VMEM((2, K), jnp.int32),
        pltpu.VMEM((K, D), jnp.float32),
        pltpu.VMEM((K, D), jnp.float32),
        pltpu.VMEM((2, D), jnp.float32),
        pltpu.VMEM((ZROWS, D), jnp.float32),
        pltpu.VMEM_SHARED((N_PAD, D), jnp.float32),
        pltpu.SemaphoreType.DMA,
        pltpu.SemaphoreType.DMA,
    ],
)
def _sc_pass2(dst_hbm, src_hbm, u_hbm, v_hbm, aff_hbm, out_hbm,
              d2, s2, ur, vr, afb, zb, out_sh, su, sv):
    cid = lax.axis_index("c")
    sid = lax.axis_index("s")
    base = (sid * NC + cid) * EPT

    pltpu.sync_copy(aff_hbm, afb)
    svec = [afb[0, pl.ds(j * 16, 16)] for j in range(8)]
    tvec = [afb[1, pl.ds(j * 16, 16)] for j in range(8)]

    zero = jnp.zeros((16,), jnp.float32)

    def zfill(i, _):
        zb[i // 8, pl.ds((i % 8) * 16, 16)] = zero
        return 0

    lax.fori_loop(0, ZROWS * 8, zfill, 0)
    for r in range(ROWS_PT // ZROWS):
        pltpu.sync_copy(zb, out_sh.at[pl.ds(sid * ROWS_PT + r * ZROWS, ZROWS)])
    plsc.subcore_barrier()

    def load_idx(ci, row):
        off = base + ci * K
        pltpu.sync_copy(dst_hbm.at[pl.ds(off, K)], d2.at[row])
        pltpu.sync_copy(src_hbm.at[pl.ds(off, K)], s2.at[row])

    load_idx(0, 0)
    pltpu.async_copy(v_hbm.at[s2.at[0]], vr, sv)

    def chunk(i, _):
        p0 = lax.rem(i, 2)
        pltpu.async_copy(u_hbm.at[d2.at[p0]], ur, su)
        load_idx(jnp.minimum(i + 1, LAST), 1 - p0)
        pltpu.make_async_copy(u_hbm.at[pl.ds(0, K)], ur, su).wait()
        pltpu.make_async_copy(v_hbm.at[pl.ds(0, K)], vr, sv).wait()

        def edge(e2, _):
            for de in range(2):
                e = e2 * 2 + de
                for j in range(8):
                    m = ur[e, pl.ds(j * 16, 16)] + vr[e, pl.ds(j * 16, 16)]
                    y = m * svec[j] + tvec[j]
                    ur[e, pl.ds(j * 16, 16)] = jnp.maximum(y, y * 0.3)
            return 0

        lax.fori_loop(0, K // 2, edge, 0)
        pltpu.async_copy(v_hbm.at[s2.at[1 - p0]], vr, sv)
        pltpu.sync_copy(ur, out_sh.at[d2.at[p0]], add=True)
        return 0

    lax.fori_loop(0, NCHUNK, chunk, 0)
    pltpu.make_async_copy(v_hbm.at[pl.ds(0, K)], vr, sv).wait()
    plsc.subcore_barrier()

    for r in range(ROWS_PT // ZROWS):
        rows = pl.ds(sid * ROWS_PT + r * ZROWS, ZROWS)
        pltpu.sync_copy(out_sh.at[rows], out_hbm.at[cid, rows])


def _tc_uv(x, wcomb):
    def body(x_ref, w_ref, u_ref, v_ref):
        xv = x_ref[...]
        u_ref[...] = jnp.dot(xv, w_ref[:, :D], preferred_element_type=jnp.float32)
        v_ref[...] = jnp.dot(xv, w_ref[:, D:], preferred_element_type=jnp.float32)

    return pl.pallas_call(
        body,
        out_shape=(
            jax.ShapeDtypeStruct((N, D), jnp.float32),
            jax.ShapeDtypeStruct((N, D), jnp.float32),
        ),
    )(x, wcomb)


def _tc_epilogue(partials, inv_cnt):
    def body(p_ref, i_ref, o_ref):
        o_ref[...] = (p_ref[0] + p_ref[1]) * i_ref[...]

    return pl.pallas_call(
        body,
        out_shape=jax.ShapeDtypeStruct((N, D), jnp.float32),
    )(partials, inv_cnt)


def kernel(feature, edge_index, W, b, gamma, beta):
    del b
    dst = edge_index[1].astype(jnp.int32)
    src = edge_index[0].astype(jnp.int32)
    dst3 = dst.reshape(NW, NCHUNK, K)
    src3 = src.reshape(NW, NCHUNK, K)

    w1 = W[:, :D]
    w2 = W[:, D:]
    wcomb = jnp.concatenate([(w1 - w2).T, w2.T], axis=1)

    u, v = _tc_uv(feature, wcomb)

    stats, cntp = _sc_pass1(dst3, src3, u, v)
    ssum = stats[:, 0, :].sum(axis=0)
    sqsum = stats[:, 1, :].sum(axis=0)
    mean = ssum / E
    var = sqsum / E - mean * mean
    scale = gamma * lax.rsqrt(var + 1e-5)
    shift = beta - mean * scale
    aff = jnp.stack([scale, shift])

    cnt = cntp.reshape(NC, CNT_PAD)[:, :N].sum(axis=0)
    inv_cnt = (1.0 / jnp.maximum(cnt, 1.0)).reshape(N, 1)

    partials = _sc_pass2(dst, src, u, v, aff)[:, :N, :]
    return _tc_epilogue(partials, inv_cnt)
